# baseline (speedup 1.0000x reference)
import functools

import jax
import jax.numpy as jnp
from jax import lax
from jax.experimental import pallas as pl
from jax.experimental.pallas import tpu as pltpu
from jax.experimental.pallas import tpu_sc as plsc

N = 10000
E = 320000
D = 128

NW = 32
NP = 320
N_PAD = NW * NP
CHUNK = 8000
NG = CHUNK // 16
NCHUNK = E // CHUNK
GB = 64
ACC_ROWS = NP + 8
DUMMY = NP

MM_BLOCK = 512
SEL_BLOCK = 1024


def _matmul_body(h_ref, w_ref, b_ref, o_ref):
    acc = lax.dot_general(h_ref[...], w_ref[...], (((1,), (1,)), ((), ())),
                          preferred_element_type=jnp.float32)
    o_ref[...] = jnp.maximum(acc + b_ref[...], 0.0)


def _select_body(a_ref, h_ref, o_ref):
    a = a_ref[...]
    o_ref[...] = jnp.where(a < 0.0, h_ref[...], a)


def _sc_body(hh_hbm, src_hbm, dst_hbm, out_hbm,
             src0, src1, dstb0, dstb1, csrc_v, cdst_v, msg0, msg1, acc_v,
             ssem0, ssem1, gsem0, gsem1):
    srcb = (src0, src1)
    dstb = (dstb0, dstb1)
    msgb = (msg0, msg1)
    ssem = (ssem0, ssem1)
    gsem = (gsem0, gsem1)

    cid = lax.axis_index("c")
    sid = lax.axis_index("s")
    wid = sid * 2 + cid
    lo = wid * NP
    hi = lo + NP

    neg1 = jnp.full((16,), -1.0, jnp.float32)

    def init_row(r, carry):
        for j in range(D // 16):
            acc_v[r, pl.ds(j * 16, 16)] = neg1
        return carry
    lax.fori_loop(0, ACC_ROWS, init_row, 0)

    def stage_issue(c, slot):
        base = pl.multiple_of(c * CHUNK, CHUNK)
        pltpu.async_copy(src_hbm.at[pl.ds(base, CHUNK)], srcb[slot], ssem[slot])
        pltpu.async_copy(dst_hbm.at[pl.ds(base, CHUNK)], dstb[slot], ssem[slot])

    def stage_drain(slot):
        pltpu.make_async_copy(src_hbm.at[pl.ds(0, CHUNK)], srcb[slot],
                              ssem[slot]).wait()
        pltpu.make_async_copy(dst_hbm.at[pl.ds(0, CHUNK)], dstb[slot],
                              ssem[slot]).wait()

    stage_issue(0, 0)
    stage_issue(1, 1)

    def process_chunk(c, slot):
        stage_drain(slot)
        sv, dv = srcb[slot], dstb[slot]

        def grp(g, n):
            s = sv[pl.ds(g * 16, 16)]
            d = dv[pl.ds(g * 16, 16)]
            m = (d >= lo) & (d < hi)
            pos = plsc.cumsum(m.astype(jnp.int32))
            idx = n + pos - 1
            plsc.store_scatter(csrc_v, [idx], s, mask=m)
            plsc.store_scatter(cdst_v, [idx], d - lo, mask=m)
            return n + pos[15]

        n = lax.fori_loop(0, NG, grp, jnp.int32(0))

        @pl.when(c + 2 < NCHUNK)
        def _():
            stage_issue(c + 2, slot)

        for t in range(GB // 16):
            csrc_v[pl.ds(n + t * 16, 16)] = jnp.zeros((16,), jnp.int32)
            cdst_v[pl.ds(n + t * 16, 16)] = jnp.full((16,), DUMMY, jnp.int32)
        nb = (n + GB - 1) // GB

        def g_issue(bidx, gslot):
            off = pl.multiple_of(bidx * GB, GB)
            pltpu.async_copy(hh_hbm.at[csrc_v.at[pl.ds(off, GB)]],
                             msgb[gslot], gsem[gslot])

        def g_drain(gslot):
            pltpu.make_async_copy(hh_hbm.at[pl.ds(0, GB)], msgb[gslot],
                                  gsem[gslot]).wait()

        def batch(bidx, carry):
            g_issue(bidx, 0)
            g_drain(0)
            off = pl.multiple_of(bidx * GB, GB)
            mref = msgb[0]
            for s2 in range(GB // 16):
                dl = cdst_v[pl.ds(off + s2 * 16, 16)]
                for e in range(16):
                    r = dl[e]
                    mrow = s2 * 16 + e
                    for j in range(D // 16):
                        sl = pl.ds(j * 16, 16)
                        acc_v[r, sl] = jnp.maximum(acc_v[r, sl],
                                                   mref[mrow, sl])
            return carry

        lax.fori_loop(0, nb, batch, 0)

    def pair(p, carry):
        process_chunk(2 * p, 0)
        process_chunk(2 * p + 1, 1)
        return carry

    lax.fori_loop(0, NCHUNK // 2, pair, 0)

    pltpu.sync_copy(acc_v.at[pl.ds(0, NP)], out_hbm.at[pl.ds(lo, NP)])


def _segment_max(hh, src, dst):
    mesh = plsc.VectorSubcoreMesh(core_axis_name="c", subcore_axis_name="s")
    run = functools.partial(
        pl.kernel, mesh=mesh,
        compiler_params=pltpu.CompilerParams(needs_layout_passes=False),
        out_type=jax.ShapeDtypeStruct((N_PAD, D), jnp.float32),
        scratch_types=[
            pltpu.VMEM((CHUNK,), jnp.int32),
            pltpu.VMEM((CHUNK,), jnp.int32),
            pltpu.VMEM((CHUNK,), jnp.int32),
            pltpu.VMEM((CHUNK,), jnp.int32),
            pltpu.VMEM((CHUNK + GB,), jnp.int32),
            pltpu.VMEM((CHUNK + GB,), jnp.int32),
            pltpu.VMEM((GB, D), jnp.float32),
            pltpu.VMEM((GB, D), jnp.float32),
            pltpu.VMEM((ACC_ROWS, D), jnp.float32),
            pltpu.SemaphoreType.DMA,
            pltpu.SemaphoreType.DMA,
            pltpu.SemaphoreType.DMA,
            pltpu.SemaphoreType.DMA,
        ],
    )(_sc_body)
    return run(hh, src, dst)


def kernel(h, edge_index, h_in, W, b):
    h_pad = jnp.pad(h, ((0, N_PAD - N), (0, 0)))
    hh = pl.pallas_call(
        _matmul_body,
        grid=(N_PAD // MM_BLOCK,),
        in_specs=[
            pl.BlockSpec((MM_BLOCK, D), lambda i: (i, 0)),
            pl.BlockSpec((D, D), lambda i: (0, 0)),
            pl.BlockSpec((1, D), lambda i: (0, 0)),
        ],
        out_specs=pl.BlockSpec((MM_BLOCK, D), lambda i: (i, 0)),
        out_shape=jax.ShapeDtypeStruct((N_PAD, D), jnp.float32),
    )(h_pad, W, b.reshape(1, D))

    agg = _segment_max(hh, edge_index[0], edge_index[1])

    out = pl.pallas_call(
        _select_body,
        grid=(N_PAD // SEL_BLOCK,),
        in_specs=[
            pl.BlockSpec((SEL_BLOCK, D), lambda i: (i, 0)),
            pl.BlockSpec((SEL_BLOCK, D), lambda i: (i, 0)),
        ],
        out_specs=pl.BlockSpec((SEL_BLOCK, D), lambda i: (i, 0)),
        out_shape=jax.ShapeDtypeStruct((N_PAD, D), jnp.float32),
    )(agg, hh)
    return out[:N]


# speedup vs baseline: 1.4115x; 1.4115x over previous
import functools

import jax
import jax.numpy as jnp
from jax import lax
from jax.experimental import pallas as pl
from jax.experimental.pallas import tpu as pltpu
from jax.experimental.pallas import tpu_sc as plsc

N = 10000
E = 320000
D = 128

NW = 32
NP = 320
N_PAD = NW * NP
CHUNK = 8000
NG = CHUNK // 16
NCHUNK = E // CHUNK
ACC_ROWS = NP + 8
DUMMY = NP

MM_BLOCK = 512
SEL_BLOCK = 1024


def _matmul_body(h_ref, w_ref, b_ref, o_ref):
    acc = lax.dot_general(h_ref[...], w_ref[...], (((1,), (1,)), ((), ())),
                          preferred_element_type=jnp.float32)
    o_ref[...] = jnp.maximum(acc + b_ref[...], 0.0)


def _select_body(a_ref, h_ref, o_ref):
    a = a_ref[...]
    o_ref[...] = jnp.where(a < 0.0, h_ref[...], a)


def _sc_body(hh_hbm, src_hbm, dst_hbm, out_hbm,
             src_v, dst_v, csrc_v, cdst_v, msg_v, acc_v, sem):
    cid = lax.axis_index("c")
    sid = lax.axis_index("s")
    wid = sid * 2 + cid
    lo = wid * NP
    hi = lo + NP

    neg1 = jnp.full((16,), -1.0, jnp.float32)

    def init_row(r, carry):
        for j in range(D // 16):
            acc_v[r, pl.ds(j * 16, 16)] = neg1
        return carry
    lax.fori_loop(0, ACC_ROWS, init_row, 0)

    def chunk_body(c, carry):
        base = c * CHUNK
        pltpu.sync_copy(src_hbm.at[pl.ds(base, CHUNK)], src_v)
        pltpu.sync_copy(dst_hbm.at[pl.ds(base, CHUNK)], dst_v)

        def grp(g, n):
            s = src_v[pl.ds(g * 16, 16)]
            d = dst_v[pl.ds(g * 16, 16)]
            m = (d >= lo) & (d < hi)
            pos = plsc.cumsum(m.astype(jnp.int32))
            idx = n + pos - 1
            plsc.store_scatter(csrc_v, [idx], s, mask=m)
            plsc.store_scatter(cdst_v, [idx], d - lo, mask=m)
            return n + pos[15]

        n = lax.fori_loop(0, NG, grp, jnp.int32(0))

        csrc_v[pl.ds(n, 16)] = jnp.zeros((16,), jnp.int32)
        cdst_v[pl.ds(n, 16)] = jnp.full((16,), DUMMY, jnp.int32)
        nb = (n + 15) // 16

        def batch(b, bcarry):
            idx = csrc_v[pl.ds(b * 16, 16)]
            cp = pltpu.async_copy(hh_hbm.at[idx], msg_v, sem)
            dl = cdst_v[pl.ds(b * 16, 16)]
            cp.wait()
            for e in range(16):
                r = dl[e]
                for j in range(D // 16):
                    sl = pl.ds(j * 16, 16)
                    acc_v[r, sl] = jnp.maximum(acc_v[r, sl], msg_v[e, sl])
            return bcarry

        lax.fori_loop(0, nb, batch, 0)
        return carry

    lax.fori_loop(0, NCHUNK, chunk_body, 0)

    pltpu.sync_copy(acc_v.at[pl.ds(0, NP)], out_hbm.at[pl.ds(lo, NP)])


def _segment_max(hh, src, dst):
    mesh = plsc.VectorSubcoreMesh(core_axis_name="c", subcore_axis_name="s")
    run = functools.partial(
        pl.kernel, mesh=mesh,
        compiler_params=pltpu.CompilerParams(needs_layout_passes=False),
        out_type=jax.ShapeDtypeStruct((N_PAD, D), jnp.float32),
        scratch_types=[
            pltpu.VMEM((CHUNK,), jnp.int32),
            pltpu.VMEM((CHUNK,), jnp.int32),
            pltpu.VMEM((CHUNK + 16,), jnp.int32),
            pltpu.VMEM((CHUNK + 16,), jnp.int32),
            pltpu.VMEM((16, D), jnp.float32),
            pltpu.VMEM((ACC_ROWS, D), jnp.float32),
            pltpu.SemaphoreType.DMA,
        ],
    )(_sc_body)
    return run(hh, src, dst)


def kernel(h, edge_index, h_in, W, b):
    h_pad = jnp.pad(h, ((0, N_PAD - N), (0, 0)))
    hh = pl.pallas_call(
        _matmul_body,
        grid=(N_PAD // MM_BLOCK,),
        in_specs=[
            pl.BlockSpec((MM_BLOCK, D), lambda i: (i, 0)),
            pl.BlockSpec((D, D), lambda i: (0, 0)),
            pl.BlockSpec((1, D), lambda i: (0, 0)),
        ],
        out_specs=pl.BlockSpec((MM_BLOCK, D), lambda i: (i, 0)),
        out_shape=jax.ShapeDtypeStruct((N_PAD, D), jnp.float32),
    )(h_pad, W, b.reshape(1, D))

    agg = _segment_max(hh, edge_index[0], edge_index[1])

    out = pl.pallas_call(
        _select_body,
        grid=(N_PAD // SEL_BLOCK,),
        in_specs=[
            pl.BlockSpec((SEL_BLOCK, D), lambda i: (i, 0)),
            pl.BlockSpec((SEL_BLOCK, D), lambda i: (i, 0)),
        ],
        out_specs=pl.BlockSpec((SEL_BLOCK, D), lambda i: (i, 0)),
        out_shape=jax.ShapeDtypeStruct((N_PAD, D), jnp.float32),
    )(agg, hh)
    return out[:N]


# speedup vs baseline: 1.4788x; 1.0477x over previous
import functools

import jax
import jax.numpy as jnp
from jax import lax
from jax.experimental import pallas as pl
from jax.experimental.pallas import tpu as pltpu
from jax.experimental.pallas import tpu_sc as plsc

N = 10000
E = 320000
D = 128

NW = 32
NP = 320
N_PAD = NW * NP
CHUNK = 16000
NG = CHUNK // 16
NCHUNK = E // CHUNK
ACC_ROWS = NP + 8
DUMMY = NP

MM_BLOCK = 512
SEL_BLOCK = 1024


def _matmul_body(h_ref, w_ref, b_ref, o_ref):
    acc = lax.dot_general(h_ref[...], w_ref[...], (((1,), (1,)), ((), ())),
                          preferred_element_type=jnp.float32)
    o_ref[...] = jnp.maximum(acc + b_ref[...], 0.0)


def _select_body(a_ref, h_ref, o_ref):
    a = a_ref[...]
    o_ref[...] = jnp.where(a < 0.0, h_ref[...], a)


def _sc_body(hh_hbm, src_hbm, dst_hbm, out_hbm,
             src_v, dst_v, csrc_v, cdst_v, msg_v, acc_v, sem):
    cid = lax.axis_index("c")
    sid = lax.axis_index("s")
    wid = sid * 2 + cid
    lo = wid * NP
    hi = lo + NP

    neg1 = jnp.full((16,), -1.0, jnp.float32)

    def init_row(r, carry):
        for j in range(D // 16):
            acc_v[r, pl.ds(j * 16, 16)] = neg1
        return carry
    lax.fori_loop(0, ACC_ROWS, init_row, 0)

    def chunk_body(c, carry):
        base = c * CHUNK
        pltpu.sync_copy(src_hbm.at[pl.ds(base, CHUNK)], src_v)
        pltpu.sync_copy(dst_hbm.at[pl.ds(base, CHUNK)], dst_v)

        def grp(g, n):
            s = src_v[pl.ds(g * 16, 16)]
            d = dst_v[pl.ds(g * 16, 16)]
            m = (d >= lo) & (d < hi)
            pos = plsc.cumsum(m.astype(jnp.int32))
            idx = n + pos - 1
            plsc.store_scatter(csrc_v, [idx], s, mask=m)
            plsc.store_scatter(cdst_v, [idx], d - lo, mask=m)
            return n + pos[15]

        n = lax.fori_loop(0, NG, grp, jnp.int32(0))

        csrc_v[pl.ds(n, 16)] = jnp.zeros((16,), jnp.int32)
        cdst_v[pl.ds(n, 16)] = jnp.full((16,), DUMMY, jnp.int32)
        nb = (n + 15) // 16

        def batch(b, bcarry):
            idx = csrc_v[pl.ds(b * 16, 16)]
            cp = pltpu.async_copy(hh_hbm.at[idx], msg_v, sem)
            dl = cdst_v[pl.ds(b * 16, 16)]
            cp.wait()
            for e in range(16):
                r = dl[e]
                for j in range(D // 16):
                    sl = pl.ds(j * 16, 16)
                    acc_v[r, sl] = jnp.maximum(acc_v[r, sl], msg_v[e, sl])
            return bcarry

        lax.fori_loop(0, nb, batch, 0)
        return carry

    lax.fori_loop(0, NCHUNK, chunk_body, 0)

    pltpu.sync_copy(acc_v.at[pl.ds(0, NP)], out_hbm.at[pl.ds(lo, NP)])


def _segment_max(hh, src, dst):
    mesh = plsc.VectorSubcoreMesh(core_axis_name="c", subcore_axis_name="s")
    run = functools.partial(
        pl.kernel, mesh=mesh,
        compiler_params=pltpu.CompilerParams(needs_layout_passes=False),
        out_type=jax.ShapeDtypeStruct((N_PAD, D), jnp.float32),
        scratch_types=[
            pltpu.VMEM((CHUNK,), jnp.int32),
            pltpu.VMEM((CHUNK,), jnp.int32),
            pltpu.VMEM((CHUNK + 16,), jnp.int32),
            pltpu.VMEM((CHUNK + 16,), jnp.int32),
            pltpu.VMEM((16, D), jnp.float32),
            pltpu.VMEM((ACC_ROWS, D), jnp.float32),
            pltpu.SemaphoreType.DMA,
        ],
    )(_sc_body)
    return run(hh, src, dst)


def kernel(h, edge_index, h_in, W, b):
    h_pad = jnp.pad(h, ((0, N_PAD - N), (0, 0)))
    hh = pl.pallas_call(
        _matmul_body,
        grid=(N_PAD // MM_BLOCK,),
        in_specs=[
            pl.BlockSpec((MM_BLOCK, D), lambda i: (i, 0)),
            pl.BlockSpec((D, D), lambda i: (0, 0)),
            pl.BlockSpec((1, D), lambda i: (0, 0)),
        ],
        out_specs=pl.BlockSpec((MM_BLOCK, D), lambda i: (i, 0)),
        out_shape=jax.ShapeDtypeStruct((N_PAD, D), jnp.float32),
    )(h_pad, W, b.reshape(1, D))

    agg = _segment_max(hh, edge_index[0], edge_index[1])

    out = pl.pallas_call(
        _select_body,
        grid=(N_PAD // SEL_BLOCK,),
        in_specs=[
            pl.BlockSpec((SEL_BLOCK, D), lambda i: (i, 0)),
            pl.BlockSpec((SEL_BLOCK, D), lambda i: (i, 0)),
        ],
        out_specs=pl.BlockSpec((SEL_BLOCK, D), lambda i: (i, 0)),
        out_shape=jax.ShapeDtypeStruct((N_PAD, D), jnp.float32),
    )(agg, hh)
    return out[:N]
